# baseline (speedup 1.0000x reference)
import jax
import jax.numpy as jnp
from jax.experimental import pallas as pl
from jax.experimental.pallas import tpu as pltpu

_B, _L, _N, _E, _D, _K = 512, 512, 8, 8, 512, 2
_TOK_BLK = 128
_ROWS = _TOK_BLK * _N
_GRID = _B // _TOK_BLK


def _moe_body(xg_ref, xm_ref, wg_ref, bg_ref, we_ref, be_ref, y_ref, loss_ref,
              gates_ref, imp_ref, load_ref):
    i = pl.program_id(0)

    gr_prev = gates_ref[...]

    xb = xg_ref[...]
    xm = jnp.sum(xb.reshape(_TOK_BLK, _N, _L), axis=1) * (1.0 / _N)
    logits = jnp.dot(xm, wg_ref[...], preferred_element_type=jnp.float32)
    logits = logits + bg_ref[...]
    logits = jnp.where(jnp.isnan(logits), 0.0, logits)
    mx = jnp.max(logits, axis=1, keepdims=True)
    p = jnp.exp(logits - mx)
    p = p / jnp.sum(p, axis=1, keepdims=True)
    iota = jax.lax.broadcasted_iota(jnp.int32, (_TOK_BLK, _E), 1)
    i1 = jnp.argmax(p, axis=1)[:, None]
    m1 = jnp.max(p, axis=1, keepdims=True)
    p2 = jnp.where(iota == i1, -1.0, p)
    i2 = jnp.argmax(p2, axis=1)[:, None]
    m2 = jnp.max(p2, axis=1, keepdims=True)
    denom = m1 + m2 + 1e-6
    gates = (jnp.where(iota == i1, m1 / denom, 0.0)
             + jnp.where(iota == i2, m2 / denom, 0.0))

    @pl.when(i == 0)
    def _():
        imp_ref[...] = jnp.zeros_like(imp_ref)
        load_ref[...] = jnp.zeros_like(load_ref)

    @pl.when(i < _GRID)
    def _():
        imp_ref[...] += jnp.sum(gates, axis=0, keepdims=True)
        load_ref[...] += jnp.sum((gates > 0).astype(jnp.float32), axis=0,
                                 keepdims=True)

    xbm = xm_ref[...]
    acc = jnp.dot(gr_prev, be_ref[...],
                  preferred_element_type=jnp.float32)
    for e in range(_E):
        pe = jnp.dot(xbm, we_ref[e], preferred_element_type=jnp.float32)
        acc = acc + gr_prev[:, e:e + 1] * pe
    y_ref[...] = acc
    gates_ref[...] = jnp.broadcast_to(
        gates[:, None, :], (_TOK_BLK, _N, _E)).reshape(_ROWS, _E)

    @pl.when(i == _GRID)
    def _():
        def cv2(v):
            mu = jnp.sum(v) * (1.0 / _E)
            var = jnp.sum((v - mu) ** 2) * (1.0 / (_E - 1))
            return var / (mu * mu + 1e-10)
        loss_ref[...] = (cv2(imp_ref[...]) + cv2(load_ref[...])).reshape(1, 1)


def _run(xt, W_gate, bg2, W_experts, b_experts, interpret=False):
    last = _GRID - 1
    return pl.pallas_call(
        _moe_body,
        grid=(_GRID + 1,),
        in_specs=[
            pl.BlockSpec((_ROWS, _L), lambda i: (jnp.minimum(i, last), 0)),
            pl.BlockSpec((_ROWS, _L), lambda i: (jnp.maximum(i - 1, 0), 0)),
            pl.BlockSpec((_L, _E), lambda i: (0, 0)),
            pl.BlockSpec((1, _E), lambda i: (0, 0)),
            pl.BlockSpec((_E, _L, _D), lambda i: (0, 0, 0)),
            pl.BlockSpec((_E, _D), lambda i: (0, 0)),
        ],
        out_specs=[
            pl.BlockSpec((_ROWS, _D), lambda i: (jnp.maximum(i - 1, 0), 0)),
            pl.BlockSpec((1, 1), lambda i: (0, 0)),
        ],
        out_shape=[
            jax.ShapeDtypeStruct((_B * _N, _D), jnp.float32),
            jax.ShapeDtypeStruct((1, 1), jnp.float32),
        ],
        scratch_shapes=[
            pltpu.VMEM((_ROWS, _E), jnp.float32),
            pltpu.VMEM((1, _E), jnp.float32),
            pltpu.VMEM((1, _E), jnp.float32),
        ],
        interpret=interpret,
    )(xt, xt, W_gate, bg2, W_experts, b_experts)


def kernel(x, W_gate, b_gate, W_experts, b_experts):
    xt = x.transpose(0, 2, 1).reshape(_B * _N, _L)
    yt, loss = _run(xt, W_gate, b_gate.reshape(1, _E), W_experts, b_experts)
    y = yt.reshape(_B, _N, _D).transpose(0, 2, 1)
    return y, loss[0, 0]


# speedup vs baseline: 1.1406x; 1.1406x over previous
import jax
import jax.numpy as jnp
from jax.experimental import pallas as pl
from jax.experimental.pallas import tpu as pltpu

_B, _L, _N, _E, _D, _K = 512, 512, 8, 8, 512, 2
_TOK_BLK = 64
_ROWS = _TOK_BLK * _N
_GRID = _B // _TOK_BLK


def _moe_body(xt_ref, wg_ref, bg_ref, we_ref, be_ref, y_ref, loss_ref,
              imp_ref, load_ref):
    i = pl.program_id(0)
    xb = xt_ref[...]

    xm = jnp.sum(xb.reshape(_TOK_BLK, _N, _L), axis=1) * (1.0 / _N)
    logits = jnp.dot(xm, wg_ref[...], preferred_element_type=jnp.float32)
    logits = logits + bg_ref[...]
    logits = jnp.where(jnp.isnan(logits), 0.0, logits)
    mx = jnp.max(logits, axis=1, keepdims=True)
    p = jnp.exp(logits - mx)
    p = p / jnp.sum(p, axis=1, keepdims=True)
    iota = jax.lax.broadcasted_iota(jnp.int32, (_TOK_BLK, _E), 1)
    i1 = jnp.argmax(p, axis=1)[:, None]
    m1 = jnp.max(p, axis=1, keepdims=True)
    p2 = jnp.where(iota == i1, -1.0, p)
    i2 = jnp.argmax(p2, axis=1)[:, None]
    m2 = jnp.max(p2, axis=1, keepdims=True)
    denom = m1 + m2 + 1e-6
    gates = (jnp.where(iota == i1, m1 / denom, 0.0)
             + jnp.where(iota == i2, m2 / denom, 0.0))

    @pl.when(i == 0)
    def _():
        imp_ref[...] = jnp.zeros_like(imp_ref)
        load_ref[...] = jnp.zeros_like(load_ref)

    imp_ref[...] += jnp.sum(gates, axis=0, keepdims=True)
    load_ref[...] += jnp.sum((gates > 0).astype(jnp.float32), axis=0,
                             keepdims=True)

    gates_rows = jnp.broadcast_to(gates[:, None, :],
                                  (_TOK_BLK, _N, _E)).reshape(_ROWS, _E)
    acc = jnp.dot(gates_rows, be_ref[...],
                  preferred_element_type=jnp.float32)
    for e in range(_E):
        pe = jnp.dot(xb, we_ref[e], preferred_element_type=jnp.float32)
        acc = acc + gates_rows[:, e:e + 1] * pe
    y_ref[...] = acc

    @pl.when(i == _GRID - 1)
    def _():
        def cv2(v):
            mu = jnp.sum(v) * (1.0 / _E)
            var = jnp.sum((v - mu) ** 2) * (1.0 / (_E - 1))
            return var / (mu * mu + 1e-10)
        loss_ref[...] = (cv2(imp_ref[...]) + cv2(load_ref[...])).reshape(1, 1)


def _run(xt, W_gate, bg2, W_experts, b_experts, interpret=False):
    return pl.pallas_call(
        _moe_body,
        grid=(_GRID,),
        in_specs=[
            pl.BlockSpec((_ROWS, _L), lambda i: (i, 0)),
            pl.BlockSpec((_L, _E), lambda i: (0, 0)),
            pl.BlockSpec((1, _E), lambda i: (0, 0)),
            pl.BlockSpec((_E, _L, _D), lambda i: (0, 0, 0)),
            pl.BlockSpec((_E, _D), lambda i: (0, 0)),
        ],
        out_specs=[
            pl.BlockSpec((_ROWS, _D), lambda i: (i, 0)),
            pl.BlockSpec((1, 1), lambda i: (0, 0)),
        ],
        out_shape=[
            jax.ShapeDtypeStruct((_B * _N, _D), jnp.float32),
            jax.ShapeDtypeStruct((1, 1), jnp.float32),
        ],
        scratch_shapes=[
            pltpu.VMEM((1, _E), jnp.float32),
            pltpu.VMEM((1, _E), jnp.float32),
        ],
        interpret=interpret,
    )(xt, W_gate, bg2, W_experts, b_experts)


def kernel(x, W_gate, b_gate, W_experts, b_experts):
    xt = x.transpose(0, 2, 1).reshape(_B * _N, _L)
    yt, loss = _run(xt, W_gate, b_gate.reshape(1, _E), W_experts, b_experts)
    y = yt.reshape(_B, _N, _D).transpose(0, 2, 1)
    return y, loss[0, 0]
